# baseline (speedup 1.0000x reference)
import functools

import jax
import jax.numpy as jnp
from jax import lax
from jax.experimental import pallas as pl
from jax.experimental.pallas import tpu as pltpu
from jax.experimental.pallas import tpu_sc as plsc

_N, _T, _D = 4, 8192, 1024
_NW = 32
_C = 8
_K = 4
_ROWS_PER_W = (_N * _T) // _NW
_N_CHUNKS = _ROWS_PER_W // _C
_N_GROUPS = _N_CHUNKS // _K


def _sc_kernel(x_hbm, pos_hbm, out_hbm, *bufs):
    xvs, pvs, ovs = bufs[0:_K], bufs[_K:2 * _K], bufs[2 * _K:3 * _K]
    sin_x, sin_p, sout = (bufs[3 * _K:4 * _K], bufs[4 * _K:5 * _K],
                          bufs[5 * _K:6 * _K])
    wid = lax.axis_index("s") * 2 + lax.axis_index("c")
    row_base = wid * _ROWS_PER_W
    t_base = row_base % _T

    def src_x(i):
        return x_hbm.at[pl.ds(row_base + i * _C, _C), :]

    def src_p(i):
        return pos_hbm.at[pl.ds(t_base + i * _C, _C), :]

    def dst_o(i):
        return out_hbm.at[pl.ds(row_base + i * _C, _C), :]

    for b in range(_K):
        pltpu.async_copy(src_x(b), xvs[b], sin_x[b])
        pltpu.async_copy(src_p(b), pvs[b], sin_p[b])

    def group_body(g, carry):
        for b in range(_K):
            j = g * _K + b
            pltpu.make_async_copy(src_x(j), xvs[b], sin_x[b]).wait()
            pltpu.make_async_copy(src_p(j), pvs[b], sin_p[b]).wait()

            @pl.when(j >= _K)
            def _():
                pltpu.make_async_copy(ovs[b], dst_o(j - _K), sout[b]).wait()

            def row_body(r, c2):
                for u in range(_D // 16):
                    sl = pl.ds(u * 16, 16)
                    ovs[b][r, sl] = xvs[b][r, sl] + pvs[b][r, sl]
                return c2

            lax.fori_loop(0, _C, row_body, 0, unroll=False)

            @pl.when(j + _K < _N_CHUNKS)
            def _():
                pltpu.async_copy(src_x(j + _K), xvs[b], sin_x[b])
                pltpu.async_copy(src_p(j + _K), pvs[b], sin_p[b])

            pltpu.async_copy(ovs[b], dst_o(j), sout[b])
        return carry

    lax.fori_loop(0, _N_GROUPS, group_body, 0, unroll=False)

    for b in range(_K):
        j = _N_CHUNKS - _K + b
        pltpu.make_async_copy(ovs[b], dst_o(j), sout[b]).wait()


@jax.jit
def _sc_add(Xf, pos_table):
    mesh = plsc.VectorSubcoreMesh(core_axis_name="c", subcore_axis_name="s")
    scratch = ([pltpu.VMEM((_C, _D), jnp.float32)] * (3 * _K)
               + [pltpu.SemaphoreType.DMA] * (3 * _K))
    kfn = functools.partial(
        pl.kernel,
        mesh=mesh,
        out_type=jax.ShapeDtypeStruct((_N * _T, _D), jnp.float32),
        scratch_types=scratch,
    )(_sc_kernel)
    return kfn(Xf, pos_table)


def kernel(X, pos_table):
    N, T, D = X.shape
    out = _sc_add(X.reshape(N * T, D), pos_table)
    return out.reshape(N, T, D)


# speedup vs baseline: 1.1228x; 1.1228x over previous
import functools

import jax
import jax.numpy as jnp
from jax import lax
from jax.experimental import pallas as pl
from jax.experimental.pallas import tpu as pltpu
from jax.experimental.pallas import tpu_sc as plsc

_N, _T, _D = 4, 8192, 1024
_TS = 2048
_NW = 32
_C = 8
_K = 4
_ROWS_PER_W = (_N * _TS) // _NW
_N_CHUNKS = _ROWS_PER_W // _C
_N_GROUPS = _N_CHUNKS // _K
_WPB = _NW // _N

_BLOCK_T = 2048


def _sc_kernel(x_hbm, pos_hbm, out_hbm, *bufs):
    xvs, pvs, ovs = bufs[0:_K], bufs[_K:2 * _K], bufs[2 * _K:3 * _K]
    sin_x, sin_p, sout = (bufs[3 * _K:4 * _K], bufs[4 * _K:5 * _K],
                          bufs[5 * _K:6 * _K])
    wid = lax.axis_index("s") * 2 + lax.axis_index("c")
    n = wid // _WPB
    t0 = (wid % _WPB) * _ROWS_PER_W
    x_base = n * _T + t0
    o_base = n * _TS + t0

    def src_x(i):
        return x_hbm.at[pl.ds(x_base + i * _C, _C), :]

    def src_p(i):
        return pos_hbm.at[pl.ds(t0 + i * _C, _C), :]

    def dst_o(i):
        return out_hbm.at[pl.ds(o_base + i * _C, _C), :]

    for b in range(_K):
        pltpu.async_copy(src_x(b), xvs[b], sin_x[b])
        pltpu.async_copy(src_p(b), pvs[b], sin_p[b])

    def group_body(g, carry):
        for b in range(_K):
            j = g * _K + b
            pltpu.make_async_copy(src_x(j), xvs[b], sin_x[b]).wait()
            pltpu.make_async_copy(src_p(j), pvs[b], sin_p[b]).wait()

            @pl.when(j >= _K)
            def _():
                pltpu.make_async_copy(ovs[b], dst_o(j - _K), sout[b]).wait()

            def row_body(r, c2):
                for u in range(_D // 16):
                    sl = pl.ds(u * 16, 16)
                    ovs[b][r, sl] = xvs[b][r, sl] + pvs[b][r, sl]
                return c2

            lax.fori_loop(0, _C, row_body, 0, unroll=False)

            @pl.when(j + _K < _N_CHUNKS)
            def _():
                pltpu.async_copy(src_x(j + _K), xvs[b], sin_x[b])
                pltpu.async_copy(src_p(j + _K), pvs[b], sin_p[b])

            pltpu.async_copy(ovs[b], dst_o(j), sout[b])
        return carry

    lax.fori_loop(0, _N_GROUPS, group_body, 0, unroll=False)

    for b in range(_K):
        j = _N_CHUNKS - _K + b
        pltpu.make_async_copy(ovs[b], dst_o(j), sout[b]).wait()


def _sc_slab(Xf, pos_table):
    mesh = plsc.VectorSubcoreMesh(core_axis_name="c", subcore_axis_name="s")
    scratch = ([pltpu.VMEM((_C, _D), jnp.float32)] * (3 * _K)
               + [pltpu.SemaphoreType.DMA] * (3 * _K))
    kfn = functools.partial(
        pl.kernel,
        mesh=mesh,
        out_type=jax.ShapeDtypeStruct((_N * _TS, _D), jnp.float32),
        scratch_types=scratch,
    )(_sc_kernel)
    return kfn(Xf, pos_table)


def _add_kernel(x_ref, pos_ref, o_ref):
    o_ref[...] = x_ref[...] + pos_ref[...]


def _tc_part(X, pos_table):
    N, T, D = X.shape
    bt = _BLOCK_T
    t_off = _TS // bt
    grid = ((T - _TS) // bt, N)
    return pl.pallas_call(
        _add_kernel,
        grid=grid,
        in_specs=[
            pl.BlockSpec((1, bt, D), lambda t, n: (n, t + t_off, 0)),
            pl.BlockSpec((bt, D), lambda t, n: (t + t_off, 0)),
        ],
        out_specs=pl.BlockSpec((1, bt, D), lambda t, n: (n, t + t_off, 0)),
        out_shape=jax.ShapeDtypeStruct((N, T, D), X.dtype),
    )(X, pos_table)


def kernel(X, pos_table):
    N, T, D = X.shape
    sc_out = _sc_slab(X.reshape(N * T, D), pos_table)
    tc_out = _tc_part(X, pos_table)
    return lax.dynamic_update_slice(
        tc_out, sc_out.reshape(N, _TS, D), (0, 0, 0))


# speedup vs baseline: 1.7382x; 1.5481x over previous
import jax
import jax.numpy as jnp
from jax.experimental import pallas as pl


_BLOCK_T = 2048


def _add_kernel(x_ref, pos_ref, o_ref):
    o_ref[...] = x_ref[...] + pos_ref[...]


def kernel(X, pos_table):
    N, T, D = X.shape
    bt = min(_BLOCK_T, T)
    grid = (T // bt, N)
    return pl.pallas_call(
        _add_kernel,
        grid=grid,
        in_specs=[
            pl.BlockSpec((1, bt, D), lambda t, n: (n, t, 0)),
            pl.BlockSpec((bt, D), lambda t, n: (t, 0)),
        ],
        out_specs=pl.BlockSpec((1, bt, D), lambda t, n: (n, t, 0)),
        out_shape=jax.ShapeDtypeStruct((N, T, D), X.dtype),
    )(X, pos_table)
